# Initial kernel scaffold; baseline (speedup 1.0000x reference)
import jax
import jax.numpy as jnp
from jax.experimental import pallas as pl


def kernel(coords_true, coords_pred, atoms_pred, atoms_true, charges_pred, charges_true, bonds_pred, bonds_true, batch, bond_aggregation_index, weights):
    raise NotImplementedError("write your pallas kernel here")


# speedup vs baseline: 10.8565x; 10.8565x over previous
import functools

import jax
import jax.numpy as jnp
from jax import lax
from jax.experimental import pallas as pl
from jax.experimental.pallas import tpu as pltpu
from jax.experimental.pallas import tpu_sc as plsc


def _row_losses(ct, cp, ap, at, chp, cht, bp, bt,
                regr_o, ace_o, cce_o, bce_o):
    d = cp[...] - ct[...]
    regr_o[...] = jnp.sum(d * d, axis=0, keepdims=True) * (1.0 / 3.0)

    def ce(logits, labels):
        m = jnp.max(logits, axis=0, keepdims=True)
        lse = jnp.log(jnp.sum(jnp.exp(logits - m), axis=0, keepdims=True)) + m
        ids = lax.broadcasted_iota(jnp.int32, logits.shape, 0)
        picked = jnp.sum(jnp.where(ids == labels, logits, 0.0),
                         axis=0, keepdims=True)
        return lse - picked

    ace_o[...] = ce(ap[...], at[...])
    cce_o[...] = ce(chp[...], cht[...])
    bce_o[...] = ce(bp[...], bt[...])


_L = 16
_NSUB = 16


def _sc_body(N, E, B,
             bce_h, agg_h, batch_h, regr_h, ace_h, cce_h, w_h, out_h,
             bsum, bcnt, seg_r, seg_a, seg_c, seg_b, seg_n, tot,
             zbuf, ones, idx16, vval, msum, mcnt, bidx, rv, av, cv, mv,
             s_r, s_a, s_c, s_b, s_n, wv, tvv, zidx):
    cid = lax.axis_index("c")
    sid = lax.axis_index("s")
    n_per = N // _NSUB
    e_rows = (E // 128) // _NSUB
    n_rows = (N // 128) // _NSUB

    for k in range(n_per // _L):
        zbuf[pl.ds(k * _L, _L)] = jnp.zeros((_L,), jnp.float32)
    for k in range(128 // _L):
        ones[pl.ds(k * _L, _L)] = jnp.ones((_L,), jnp.float32)

    @pl.when(cid == 0)
    def _():
        pltpu.sync_copy(zbuf, bsum.at[pl.ds(sid * n_per, n_per)])
        pltpu.sync_copy(zbuf, bcnt.at[pl.ds(sid * n_per, n_per)])

        @pl.when(sid == 0)
        def _():
            for ref in (seg_r, seg_a, seg_c, seg_b, seg_n):
                pltpu.sync_copy(zbuf.at[pl.ds(0, B)], ref)
            pltpu.sync_copy(zbuf.at[pl.ds(0, _L)], tot)

    plsc.subcore_barrier()

    @pl.when(cid == 0)
    def _():
        pltpu.sync_copy(agg_h.at[pl.ds(sid * e_rows, e_rows)], idx16)
        pltpu.sync_copy(bce_h.at[pl.ds(sid * e_rows, e_rows)], vval)
        for r in range(e_rows):
            pltpu.sync_copy(vval.at[r], bsum.at[idx16.at[r]], add=True)
            pltpu.sync_copy(ones, bcnt.at[idx16.at[r]], add=True)

    plsc.subcore_barrier()

    @pl.when(cid == 0)
    def _():
        pltpu.sync_copy(bsum.at[pl.ds(sid * n_per, n_per)], msum)
        pltpu.sync_copy(bcnt.at[pl.ds(sid * n_per, n_per)], mcnt)
        pltpu.sync_copy(batch_h.at[pl.ds(sid * n_rows, n_rows)], bidx)
        pltpu.sync_copy(regr_h.at[pl.ds(sid * n_rows, n_rows)], rv)
        pltpu.sync_copy(ace_h.at[pl.ds(sid * n_rows, n_rows)], av)
        pltpu.sync_copy(cce_h.at[pl.ds(sid * n_rows, n_rows)], cv)
        for k in range(n_per // _L):
            s = msum[pl.ds(k * _L, _L)]
            c = mcnt[pl.ds(k * _L, _L)]
            m = jnp.where(c > 0, 0.5 * s / jnp.maximum(c, 1.0), 0.0)
            mv[(k * _L) // 128, pl.ds((k * _L) % 128, _L)] = m
        for r in range(n_rows):
            pltpu.sync_copy(mv.at[r], seg_b.at[bidx.at[r]], add=True)
            pltpu.sync_copy(rv.at[r], seg_r.at[bidx.at[r]], add=True)
            pltpu.sync_copy(av.at[r], seg_a.at[bidx.at[r]], add=True)
            pltpu.sync_copy(cv.at[r], seg_c.at[bidx.at[r]], add=True)
            pltpu.sync_copy(ones, seg_n.at[bidx.at[r]], add=True)

    plsc.subcore_barrier()

    @pl.when((cid == 0) & (sid == 0))
    def _():
        pltpu.sync_copy(seg_r, s_r)
        pltpu.sync_copy(seg_a, s_a)
        pltpu.sync_copy(seg_c, s_c)
        pltpu.sync_copy(seg_b, s_b)
        pltpu.sync_copy(seg_n, s_n)
        pltpu.sync_copy(w_h, wv)
        acc_r = jnp.zeros((_L,), jnp.float32)
        acc_a = jnp.zeros((_L,), jnp.float32)
        acc_c = jnp.zeros((_L,), jnp.float32)
        acc_b = jnp.zeros((_L,), jnp.float32)
        for k in range(B // _L):
            sl = pl.ds(k * _L, _L)
            cn = s_n[sl]
            w = wv[sl]
            good = cn > 0
            cd = jnp.maximum(cn, 1.0)
            acc_r += jnp.where(good, s_r[sl] / cd, 0.0) * w
            acc_a += jnp.where(good, s_a[sl] / cd, 0.0) * w
            acc_c += jnp.where(good, s_c[sl] / cd, 0.0) * w
            acc_b += jnp.where(good, s_b[sl] / cd, 0.0) * w
        for x, acc in enumerate((acc_r, acc_a, acc_c, acc_b)):
            tvv[x, pl.ds(0, _L)] = acc
            zidx[x, pl.ds(0, _L)] = jnp.full((_L,), x, jnp.int32)
        for x in range(4):
            pltpu.sync_copy(tvv.at[x], tot.at[zidx.at[x]], add=True)
        pltpu.sync_copy(tot, out_h)


def _make_sc_kernel(N, E, B):
    mesh = plsc.VectorSubcoreMesh(core_axis_name="c", subcore_axis_name="s")
    n_per = N // _NSUB
    e_rows = (E // 128) // _NSUB
    n_rows = (N // 128) // _NSUB
    return pl.kernel(
        functools.partial(_sc_body, N, E, B),
        out_type=jax.ShapeDtypeStruct((_L,), jnp.float32),
        mesh=mesh,
        scratch_types=[
            pltpu.VMEM_SHARED((N,), jnp.float32),
            pltpu.VMEM_SHARED((N,), jnp.float32),
            pltpu.VMEM_SHARED((B,), jnp.float32),
            pltpu.VMEM_SHARED((B,), jnp.float32),
            pltpu.VMEM_SHARED((B,), jnp.float32),
            pltpu.VMEM_SHARED((B,), jnp.float32),
            pltpu.VMEM_SHARED((B,), jnp.float32),
            pltpu.VMEM_SHARED((_L,), jnp.float32),
            pltpu.VMEM((n_per,), jnp.float32),
            pltpu.VMEM((128,), jnp.float32),
            pltpu.VMEM((e_rows, 128), jnp.int32),
            pltpu.VMEM((e_rows, 128), jnp.float32),
            pltpu.VMEM((n_per,), jnp.float32),
            pltpu.VMEM((n_per,), jnp.float32),
            pltpu.VMEM((n_rows, 128), jnp.int32),
            pltpu.VMEM((n_rows, 128), jnp.float32),
            pltpu.VMEM((n_rows, 128), jnp.float32),
            pltpu.VMEM((n_rows, 128), jnp.float32),
            pltpu.VMEM((n_rows, 128), jnp.float32),
            pltpu.VMEM((B,), jnp.float32),
            pltpu.VMEM((B,), jnp.float32),
            pltpu.VMEM((B,), jnp.float32),
            pltpu.VMEM((B,), jnp.float32),
            pltpu.VMEM((B,), jnp.float32),
            pltpu.VMEM((B,), jnp.float32),
            pltpu.VMEM((4, _L), jnp.float32),
            pltpu.VMEM((4, _L), jnp.int32),
        ],
    )


def kernel(coords_true, coords_pred, atoms_pred, atoms_true,
           charges_pred, charges_true, bonds_pred, bonds_true,
           batch, bond_aggregation_index, weights):
    N = coords_true.shape[0]
    E = bonds_pred.shape[0]
    B = weights.shape[0]

    regr, ace, cce, bce = pl.pallas_call(
        _row_losses,
        out_shape=[
            jax.ShapeDtypeStruct((1, N), jnp.float32),
            jax.ShapeDtypeStruct((1, N), jnp.float32),
            jax.ShapeDtypeStruct((1, N), jnp.float32),
            jax.ShapeDtypeStruct((1, E), jnp.float32),
        ],
    )(
        coords_true.T, coords_pred.T,
        atoms_pred.T, atoms_true.astype(jnp.int32).reshape(1, N),
        charges_pred.T, charges_true.astype(jnp.int32).reshape(1, N),
        bonds_pred.T, bonds_true.astype(jnp.int32).reshape(1, E),
    )

    out = _make_sc_kernel(N, E, B)(
        bce.reshape(E // 128, 128),
        bond_aggregation_index.astype(jnp.int32).reshape(E // 128, 128),
        batch.astype(jnp.int32).reshape(N // 128, 128),
        regr.reshape(N // 128, 128),
        ace.reshape(N // 128, 128),
        cce.reshape(N // 128, 128),
        weights,
    )
    return (out[0], out[1], out[2], out[3])
